# baseline (speedup 1.0000x reference)
import functools

import jax
import jax.numpy as jnp
from jax import lax
from jax.experimental import pallas as pl
from jax.experimental.pallas import tpu as pltpu
from jax.experimental.pallas import tpu_sc as plsc

MAX_DEG = 10
NPD = 5000
NA = NPD * (MAX_DEG + 1)
D = 128
B = 64
NT = 32
RPT = 160
PADN = NT * RPT
BLK = NPD
NBLK = NA // BLK

_NPASS = sum(range(1, MAX_DEG + 1))


def _sc_body(atoms_hbm, idx_hbm, out_hbm, idxv, acc0, acc1, buf0, buf1, buf2,
             s_idx, s_b0, s_b1, s_o0, s_o1):
    wid = (lax.axis_index("s") * 2 + lax.axis_index("c")) ^ 1
    pltpu.sync_copy(idx_hbm.at[pl.ds(wid * (_NPASS * RPT), _NPASS * RPT)], idxv)

    accs = (acc0, acc1)
    bufs, b_sems = (buf0, buf1, buf2), (s_b0, s_b1, s_idx)
    o_sems = (s_o0, s_o1)

    sched = []
    for k, (d, j) in enumerate(
            (d, j) for d in range(1, MAX_DEG + 1) for j in range(d)):
        sched.append((d, j, bufs[k % 3], b_sems[k % 3]))

    def fire(k):
        _, _, dst, sem = sched[k]
        p = k * RPT
        h1 = pltpu.async_copy(atoms_hbm.at[idxv.at[pl.ds(p, 128)]],
                              dst.at[pl.ds(0, 128)], sem)
        h2 = pltpu.async_copy(atoms_hbm.at[idxv.at[pl.ds(p + 128, RPT - 128)]],
                              dst.at[pl.ds(128, RPT - 128)], sem)
        return (h1, h2)

    base = wid * RPT
    out_handles = {}
    pending = {0: fire(0), 1: fire(1)}
    for k in range(_NPASS):
        d, j, dst, _ = sched[k]
        if k + 2 < _NPASS:
            pending[k + 2] = fire(k + 2)
        h1, h2 = pending.pop(k)
        h1.wait()
        h2.wait()
        acc = accs[d % 2]
        if j == 0 and (d - 2) in out_handles:
            out_handles.pop(d - 2).wait()
        first = j == 0

        @plsc.parallel_loop(0, RPT, unroll=2)
        def _add(r):
            for c in range(D // 16):
                sl = pl.ds(c * 16, 16)
                if first:
                    acc[r, sl] = dst[r, sl]
                else:
                    acc[r, sl] = acc[r, sl] + dst[r, sl]

        if j == d - 1:
            out_handles[d] = pltpu.async_copy(
                accs[d % 2], out_hbm.at[d - 1, pl.ds(base, RPT)], o_sems[d % 2])
    for h in out_handles.values():
        h.wait()


def _tc_body_a(x_ref, wa_ref, bias_ref, mem_ref, wg_ref, bg_ref,
               o_ref, od_ref, acc_ref):
    i = pl.program_id(0)
    x = x_ref[...]
    o_ref[...] = (jnp.dot(x, wa_ref[0], preferred_element_type=jnp.float32)
                  + bias_ref[0])

    @pl.when(i == 0)
    def _init():
        acc_ref[...] = jnp.zeros_like(acc_ref)

    mem = mem_ref[0, 0, :]
    seg = lax.broadcasted_iota(jnp.int32, (B, BLK), 0)
    onehot = (seg == mem[None, :]).astype(jnp.float32)
    acc_ref[...] += jnp.dot(onehot, x, preferred_element_type=jnp.float32)

    @pl.when(i == NBLK - 1)
    def _fin():
        od_ref[...] = (jnp.dot(acc_ref[...], wg_ref[...],
                               preferred_element_type=jnp.float32) + bg_ref[...])


def _neighbor_sums(atoms, idx_all):
    mesh = plsc.VectorSubcoreMesh(core_axis_name="c", subcore_axis_name="s")
    f = pl.kernel(
        _sc_body,
        mesh=mesh,
        out_type=jax.ShapeDtypeStruct((MAX_DEG, PADN, D), jnp.float32),
        scratch_types=[
            pltpu.VMEM((_NPASS * RPT,), jnp.int32),
            pltpu.VMEM((RPT, D), jnp.float32),
            pltpu.VMEM((RPT, D), jnp.float32),
            pltpu.VMEM((RPT, D), jnp.float32),
            pltpu.VMEM((RPT, D), jnp.float32),
            pltpu.VMEM((RPT, D), jnp.float32),
        ] + [pltpu.SemaphoreType.DMA] * 5,
    )
    return f(atoms, idx_all)


def _tc_body_b(s_ref, t_ref, wb_ref, o_ref):
    t = t_ref[0][:BLK]
    o_ref[...] = s_ref[...] + jnp.dot(t, wb_ref[0],
                                      preferred_element_type=jnp.float32)


def _dense_a(atoms, wa, bias, mem3, wg, bg):
    return pl.pallas_call(
        _tc_body_a,
        grid=(NBLK,),
        in_specs=[
            pl.BlockSpec((BLK, D), lambda i: (i, 0)),
            pl.BlockSpec((1, D, D), lambda i: (i, 0, 0)),
            pl.BlockSpec((1, 1, D), lambda i: (i, 0, 0)),
            pl.BlockSpec((1, 1, BLK), lambda i: (i, 0, 0)),
            pl.BlockSpec((D, D), lambda i: (0, 0)),
            pl.BlockSpec((1, D), lambda i: (0, 0)),
        ],
        out_specs=[
            pl.BlockSpec((BLK, D), lambda i: (i, 0)),
            pl.BlockSpec((B, D), lambda i: (0, 0)),
        ],
        out_shape=[
            jax.ShapeDtypeStruct((NA, D), jnp.float32),
            jax.ShapeDtypeStruct((B, D), jnp.float32),
        ],
        scratch_shapes=[pltpu.VMEM((B, D), jnp.float32)],
    )(atoms, wa, bias, mem3, wg, bg)


def _dense_b(act_self, nsum, wb):
    return pl.pallas_call(
        _tc_body_b,
        grid=(MAX_DEG,),
        in_specs=[
            pl.BlockSpec((BLK, D), lambda i: (i + 1, 0)),
            pl.BlockSpec((1, PADN, D), lambda i: (i, 0, 0)),
            pl.BlockSpec((1, D, D), lambda i: (i, 0, 0)),
        ],
        out_specs=pl.BlockSpec((BLK, D), lambda i: (i, 0)),
        out_shape=jax.ShapeDtypeStruct((NA - NPD, D), jnp.float32),
    )(act_self, nsum, wb)


def kernel(atom_features, deg_slice, membership, deg_adj_list_1,
           deg_adj_list_2, deg_adj_list_3, deg_adj_list_4, deg_adj_list_5,
           deg_adj_list_6, deg_adj_list_7, deg_adj_list_8, deg_adj_list_9,
           deg_adj_list_10, W_list, b_list, batch_size, add_time):
    dals = [deg_adj_list_1, deg_adj_list_2, deg_adj_list_3, deg_adj_list_4,
            deg_adj_list_5, deg_adj_list_6, deg_adj_list_7, deg_adj_list_8,
            deg_adj_list_9, deg_adj_list_10]
    rows = []
    for dal in dals:
        rows.append(jnp.pad(dal.T, ((0, 0), (0, PADN - NPD))))
    idx_all = (jnp.concatenate(rows, axis=0)
               .reshape(_NPASS, NT, RPT)
               .transpose(1, 0, 2)
               .reshape(-1))

    nsum = _neighbor_sums(atom_features, idx_all)

    wa = jnp.stack([W_list[11]] + [W_list[0]] * MAX_DEG)
    wb = jnp.stack([W_list[d] for d in range(1, MAX_DEG + 1)])
    bias = jnp.stack([b_list[11]]
                     + [b_list[d] + b_list[0]
                        for d in range(1, MAX_DEG + 1)])[:, None, :]
    mem3 = membership.reshape(NBLK, 1, BLK)
    wg = W_list[12]
    bg = b_list[12].reshape(1, D)

    act_self, dummy = _dense_a(atom_features, wa, bias, mem3, wg, bg)
    act_rel = _dense_b(act_self, nsum, wb)
    return jnp.concatenate([act_self[:NPD], act_rel, dummy], axis=0)


# speedup vs baseline: 2.6327x; 2.6327x over previous
import functools

import jax
import jax.numpy as jnp
from jax import lax
from jax.experimental import pallas as pl
from jax.experimental.pallas import tpu as pltpu
from jax.experimental.pallas import tpu_sc as plsc

MAX_DEG = 10
NPD = 5000
NA = NPD * (MAX_DEG + 1)
D = 128
B = 64
NT = 32
RPT = 160
PADN = NT * RPT
BLK = NPD
NBLK = NA // BLK

_NPASS = sum(range(1, MAX_DEG + 1))


def _sc_body(atoms_hbm, idx_hbm, out_hbm, idxv, acc0, acc1, buf0, buf1, buf2,
             s_idx, s_b0, s_b1, s_o0, s_o1):
    wid = lax.axis_index("s") * 2 + lax.axis_index("c")
    pltpu.sync_copy(idx_hbm.at[pl.ds(wid * (_NPASS * RPT), _NPASS * RPT)], idxv)

    accs = (acc0, acc1)
    bufs, b_sems = (buf0, buf1, buf2), (s_b0, s_b1, s_idx)
    o_sems = (s_o0, s_o1)

    sched = []
    for k, (d, j) in enumerate(
            (d, j) for d in range(1, MAX_DEG + 1) for j in range(d)):
        sched.append((d, j, bufs[k % 3], b_sems[k % 3]))

    def fire(k):
        _, _, dst, sem = sched[k]
        p = k * RPT
        h1 = pltpu.async_copy(atoms_hbm.at[idxv.at[pl.ds(p, 128)]],
                              dst.at[pl.ds(0, 128)], sem)
        h2 = pltpu.async_copy(atoms_hbm.at[idxv.at[pl.ds(p + 128, RPT - 128)]],
                              dst.at[pl.ds(128, RPT - 128)], sem)
        return (h1, h2)

    base = wid * RPT
    out_handles = {}
    pending = {0: fire(0), 1: fire(1)}
    for k in range(_NPASS):
        d, j, dst, _ = sched[k]
        if k + 2 < _NPASS:
            pending[k + 2] = fire(k + 2)
        h1, h2 = pending.pop(k)
        h1.wait()
        h2.wait()
        acc = accs[d % 2]
        if j == 0 and (d - 2) in out_handles:
            out_handles.pop(d - 2).wait()
        first = j == 0

        @plsc.parallel_loop(0, RPT, unroll=2)
        def _add(r):
            for c in range(D // 16):
                sl = pl.ds(c * 16, 16)
                if first:
                    acc[r, sl] = dst[r, sl]
                else:
                    acc[r, sl] = acc[r, sl] + dst[r, sl]

        if j == d - 1:
            out_handles[d] = pltpu.async_copy(
                accs[d % 2], out_hbm.at[d - 1, pl.ds(base, RPT)], o_sems[d % 2])
    for h in out_handles.values():
        h.wait()


def _tc_body_a(x_ref, wa_ref, bias_ref, mem_ref, wg_ref, bg_ref,
               o_ref, od_ref, acc_ref):
    i = pl.program_id(0)
    x = x_ref[...]
    o_ref[...] = (jnp.dot(x, wa_ref[0], preferred_element_type=jnp.float32)
                  + bias_ref[0])

    @pl.when(i == 0)
    def _init():
        acc_ref[...] = jnp.zeros_like(acc_ref)

    mem = mem_ref[0, 0, :]
    seg = lax.broadcasted_iota(jnp.int32, (B, BLK), 0)
    onehot = (seg == mem[None, :]).astype(jnp.float32)
    acc_ref[...] += jnp.dot(onehot, x, preferred_element_type=jnp.float32)

    @pl.when(i == NBLK - 1)
    def _fin():
        od_ref[...] = (jnp.dot(acc_ref[...], wg_ref[...],
                               preferred_element_type=jnp.float32) + bg_ref[...])


def _neighbor_sums(atoms, idx_all):
    mesh = plsc.VectorSubcoreMesh(core_axis_name="c", subcore_axis_name="s")
    f = pl.kernel(
        _sc_body,
        mesh=mesh,
        out_type=jax.ShapeDtypeStruct((MAX_DEG, PADN, D), jnp.float32),
        scratch_types=[
            pltpu.VMEM((_NPASS * RPT,), jnp.int32),
            pltpu.VMEM((RPT, D), jnp.float32),
            pltpu.VMEM((RPT, D), jnp.float32),
            pltpu.VMEM((RPT, D), jnp.float32),
            pltpu.VMEM((RPT, D), jnp.float32),
            pltpu.VMEM((RPT, D), jnp.float32),
        ] + [pltpu.SemaphoreType.DMA] * 5,
    )
    return f(atoms, idx_all)


def _tc_body_b(s_ref, t_ref, wb_ref, o_ref):
    t = t_ref[0][:BLK]
    o_ref[...] = s_ref[...] + jnp.dot(t, wb_ref[0],
                                      preferred_element_type=jnp.float32)


def _dense_a(atoms, wa, bias, mem3, wg, bg):
    return pl.pallas_call(
        _tc_body_a,
        grid=(NBLK,),
        in_specs=[
            pl.BlockSpec((BLK, D), lambda i: (i, 0)),
            pl.BlockSpec((1, D, D), lambda i: (i, 0, 0)),
            pl.BlockSpec((1, 1, D), lambda i: (i, 0, 0)),
            pl.BlockSpec((1, 1, BLK), lambda i: (i, 0, 0)),
            pl.BlockSpec((D, D), lambda i: (0, 0)),
            pl.BlockSpec((1, D), lambda i: (0, 0)),
        ],
        out_specs=[
            pl.BlockSpec((BLK, D), lambda i: (i, 0)),
            pl.BlockSpec((B, D), lambda i: (0, 0)),
        ],
        out_shape=[
            jax.ShapeDtypeStruct((NA, D), jnp.float32),
            jax.ShapeDtypeStruct((B, D), jnp.float32),
        ],
        scratch_shapes=[pltpu.VMEM((B, D), jnp.float32)],
    )(atoms, wa, bias, mem3, wg, bg)


def _dense_b(act_self, nsum, wb):
    return pl.pallas_call(
        _tc_body_b,
        grid=(MAX_DEG,),
        in_specs=[
            pl.BlockSpec((BLK, D), lambda i: (i + 1, 0)),
            pl.BlockSpec((1, PADN, D), lambda i: (i, 0, 0)),
            pl.BlockSpec((1, D, D), lambda i: (i, 0, 0)),
        ],
        out_specs=pl.BlockSpec((BLK, D), lambda i: (i, 0)),
        out_shape=jax.ShapeDtypeStruct((NA - NPD, D), jnp.float32),
    )(act_self, nsum, wb)


def kernel(atom_features, deg_slice, membership, deg_adj_list_1,
           deg_adj_list_2, deg_adj_list_3, deg_adj_list_4, deg_adj_list_5,
           deg_adj_list_6, deg_adj_list_7, deg_adj_list_8, deg_adj_list_9,
           deg_adj_list_10, W_list, b_list, batch_size, add_time):
    dals = [deg_adj_list_1, deg_adj_list_2, deg_adj_list_3, deg_adj_list_4,
            deg_adj_list_5, deg_adj_list_6, deg_adj_list_7, deg_adj_list_8,
            deg_adj_list_9, deg_adj_list_10]
    padv = jnp.broadcast_to((jnp.arange(PADN - NPD, dtype=jnp.int32) * 457)
                            % NA, (1, PADN - NPD))
    rows = []
    for dal in dals:
        rows.append(jnp.concatenate(
            [dal.T, jnp.broadcast_to(padv, (dal.shape[1], PADN - NPD))], axis=1))
    idx_all = (jnp.concatenate(rows, axis=0)
               .reshape(_NPASS, NT, RPT)
               .transpose(1, 0, 2)
               .reshape(-1))

    nsum = _neighbor_sums(atom_features, idx_all)

    wa = jnp.stack([W_list[11]] + [W_list[0]] * MAX_DEG)
    wb = jnp.stack([W_list[d] for d in range(1, MAX_DEG + 1)])
    bias = jnp.stack([b_list[11]]
                     + [b_list[d] + b_list[0]
                        for d in range(1, MAX_DEG + 1)])[:, None, :]
    mem3 = membership.reshape(NBLK, 1, BLK)
    wg = W_list[12]
    bg = b_list[12].reshape(1, D)

    act_self, dummy = _dense_a(atom_features, wa, bias, mem3, wg, bg)
    act_rel = _dense_b(act_self, nsum, wb)
    return jnp.concatenate([act_self[:NPD], act_rel, dummy], axis=0)
